# baseline (speedup 1.0000x reference)
import functools

import jax
import jax.numpy as jnp
from jax import lax
from jax.experimental import pallas as pl
from jax.experimental.pallas import tpu as pltpu
from jax.experimental.pallas import tpu_sc as plsc

VOCAB = 100000
HIDDEN = 64
N_TOK = 512

_NC, _NS = 2, 16
_NW = _NC * _NS
_B_PER_W = N_TOK // _NW

VBLK = 4096
TBLK = 2048


def _repack_kernel(tt_ref, o_ref):
    t = tt_ref[...]
    eye = jnp.eye(HIDDEN, dtype=jnp.float32)
    tT = lax.dot_general(
        t, eye, (((0,), (0,)), ((), ())),
        precision=lax.Precision.HIGHEST,
        preferred_element_type=jnp.float32,
    )
    o_ref[:, :HIDDEN] = tT
    o_ref[:, HIDDEN:] = jnp.zeros((TBLK, HIDDEN), jnp.float32)


@functools.cache
def _make_sc_gather():
    mesh = plsc.VectorSubcoreMesh(core_axis_name="c", subcore_axis_name="s")

    @functools.partial(
        pl.kernel,
        mesh=mesh,
        out_type=jax.ShapeDtypeStruct((N_TOK, 2 * HIDDEN), jnp.float32),
        scratch_types=[
            pltpu.VMEM((_B_PER_W,), jnp.int32),
            pltpu.VMEM((_B_PER_W, 2 * HIDDEN), jnp.float32),
            pltpu.SemaphoreType.DMA,
        ],
    )
    def gather_kernel(table_hbm, idx_hbm, out_hbm, idx_v, rows_v, sem):
        wid = lax.axis_index("s") * _NC + lax.axis_index("c")
        base = wid * _B_PER_W
        pltpu.sync_copy(idx_hbm.at[pl.ds(base, _B_PER_W)], idx_v)
        pltpu.async_copy(table_hbm.at[idx_v], rows_v, sem).wait()
        pltpu.sync_copy(rows_v, out_hbm.at[pl.ds(base, _B_PER_W)])

    return gather_kernel


def _head_kernel(h2_ref, wt_ref, b_ref, o_ref):
    h = h2_ref[:, :HIDDEN].astype(jnp.bfloat16)
    o_ref[...] = lax.dot_general(
        h,
        wt_ref[...].astype(jnp.bfloat16),
        (((1,), (0,)), ((), ())),
        preferred_element_type=jnp.float32,
    ) + b_ref[...]


def kernel(input_ids, attention_mask, emb_table, W_head, b_head):
    del attention_mask
    ids = input_ids.reshape(N_TOK).astype(jnp.int32)

    tt_emb = emb_table.T
    tp = pl.pallas_call(
        _repack_kernel,
        grid=(pl.cdiv(VOCAB, TBLK),),
        in_specs=[pl.BlockSpec((HIDDEN, TBLK), lambda i: (0, i))],
        out_specs=pl.BlockSpec((TBLK, 2 * HIDDEN), lambda i: (i, 0)),
        out_shape=jax.ShapeDtypeStruct((VOCAB, 2 * HIDDEN), jnp.float32),
    )(tt_emb)

    h2 = _make_sc_gather()(tp, ids)

    wt = W_head.T
    b2 = b_head.reshape(1, VOCAB)
    grid = (pl.cdiv(VOCAB, VBLK),)
    logits = pl.pallas_call(
        _head_kernel,
        grid=grid,
        in_specs=[
            pl.BlockSpec((N_TOK, 2 * HIDDEN), lambda j: (0, 0)),
            pl.BlockSpec((HIDDEN, VBLK), lambda j: (0, j)),
            pl.BlockSpec((1, VBLK), lambda j: (0, j)),
        ],
        out_specs=pl.BlockSpec((N_TOK, VBLK), lambda j: (0, j)),
        out_shape=jax.ShapeDtypeStruct((N_TOK, VOCAB), jnp.float32),
    )(h2, wt, b2)

    return logits.reshape(input_ids.shape[0], input_ids.shape[1], VOCAB)


# speedup vs baseline: 1.1381x; 1.1381x over previous
import functools

import jax
import jax.numpy as jnp
from jax import lax
from jax.experimental import pallas as pl
from jax.experimental.pallas import tpu as pltpu
from jax.experimental.pallas import tpu_sc as plsc

VOCAB = 100000
HIDDEN = 64
N_TOK = 512

_NC, _NS = 2, 16
_NW = _NC * _NS
_B_PER_W = N_TOK // _NW

VBLK = 4096
TBLK = 1024
KSPLIT = 50176


def _repack_kernel(t1_ref, t2_ref, o_ref):
    eye = jnp.eye(HIDDEN, dtype=jnp.bfloat16)

    def tr(t_ref):
        return lax.dot_general(
            t_ref[...].astype(jnp.bfloat16), eye,
            (((0,), (0,)), ((), ())),
            preferred_element_type=jnp.float32,
        )

    o_ref[:, :HIDDEN] = tr(t1_ref)
    o_ref[:, HIDDEN:] = tr(t2_ref)


def _head_kernel(h2_ref, par_ref, wt_ref, b_ref, o_ref):
    par = par_ref[...]
    h = jnp.where(par > 0.5, h2_ref[:, HIDDEN:], h2_ref[:, :HIDDEN])
    o_ref[...] = lax.dot_general(
        h.astype(jnp.bfloat16),
        wt_ref[...].astype(jnp.bfloat16),
        (((1,), (0,)), ((), ())),
        preferred_element_type=jnp.float32,
    ) + b_ref[...]


@functools.cache
def _make_sc_gather():
    mesh = plsc.VectorSubcoreMesh(core_axis_name="c", subcore_axis_name="s")

    @functools.partial(
        pl.kernel,
        mesh=mesh,
        out_type=jax.ShapeDtypeStruct((N_TOK, 2 * HIDDEN), jnp.float32),
        scratch_types=[
            pltpu.VMEM((_B_PER_W,), jnp.int32),
            pltpu.VMEM((_B_PER_W, 2 * HIDDEN), jnp.float32),
            pltpu.SemaphoreType.DMA,
        ],
    )
    def gather_kernel(table_hbm, idx_hbm, out_hbm, idx_v, rows_v, sem):
        wid = lax.axis_index("s") * _NC + lax.axis_index("c")
        base = wid * _B_PER_W
        pltpu.sync_copy(idx_hbm.at[pl.ds(base, _B_PER_W)], idx_v)
        pltpu.async_copy(table_hbm.at[idx_v], rows_v, sem).wait()
        pltpu.sync_copy(rows_v, out_hbm.at[pl.ds(base, _B_PER_W)])

    return gather_kernel


def kernel(input_ids, attention_mask, emb_table, W_head, b_head):
    del attention_mask
    ids = input_ids.reshape(N_TOK).astype(jnp.int32)
    hi = ids >= KSPLIT
    ids_row = jnp.where(hi, ids - KSPLIT, ids)
    par = hi.astype(jnp.float32).reshape(N_TOK, 1)

    tt_emb = emb_table.T
    table2 = pl.pallas_call(
        _repack_kernel,
        grid=(KSPLIT // TBLK,),
        in_specs=[
            pl.BlockSpec((HIDDEN, TBLK), lambda i: (0, i)),
            pl.BlockSpec((HIDDEN, TBLK), lambda i: (0, i + KSPLIT // TBLK)),
        ],
        out_specs=pl.BlockSpec((TBLK, 2 * HIDDEN), lambda i: (i, 0)),
        out_shape=jax.ShapeDtypeStruct((KSPLIT, 2 * HIDDEN), jnp.float32),
    )(tt_emb, tt_emb)

    h2 = _make_sc_gather()(table2, ids_row)

    wt = W_head.T
    b2 = b_head.reshape(1, VOCAB)
    grid = (pl.cdiv(VOCAB, VBLK),)
    logits = pl.pallas_call(
        _head_kernel,
        grid=grid,
        in_specs=[
            pl.BlockSpec((N_TOK, 2 * HIDDEN), lambda j: (0, 0)),
            pl.BlockSpec((N_TOK, 1), lambda j: (0, 0)),
            pl.BlockSpec((HIDDEN, VBLK), lambda j: (0, j)),
            pl.BlockSpec((1, VBLK), lambda j: (0, j)),
        ],
        out_specs=pl.BlockSpec((N_TOK, VBLK), lambda j: (0, j)),
        out_shape=jax.ShapeDtypeStruct((N_TOK, VOCAB), jnp.float32),
    )(h2, par, wt, b2)

    return logits.reshape(input_ids.shape[0], input_ids.shape[1], VOCAB)
